# baseline (speedup 1.0000x reference)
import functools

import jax
import jax.numpy as jnp
from jax.experimental import pallas as pl
from jax.experimental.pallas import tpu as pltpu
from jax.experimental.pallas import tpu_sc as plsc


def _last_relevant_sc(lstm, seqlens, B, T, D):
    mesh = plsc.ScalarSubcoreMesh(axis_name="c", num_cores=2)

    @functools.partial(
        pl.kernel,
        mesh=mesh,
        out_type=jax.ShapeDtypeStruct((B, D), jnp.float32),
        scratch_types=[
            pltpu.SMEM((B,), jnp.int32),
            pltpu.SemaphoreType.DMA,
        ],
    )
    def body(lstm_hbm, seq_hbm, out_hbm, seq_s, sem):
        from jax import lax

        cid = lax.axis_index("c")
        half = B // 2
        base = cid * half
        pltpu.sync_copy(seq_hbm, seq_s)
        for i in range(half):
            b = base + i
            t = seq_s[b] - 1
            pltpu.async_copy(lstm_hbm.at[b, t], out_hbm.at[b], sem)
        pltpu.make_async_copy(
            lstm_hbm.at[0, pl.ds(0, half)], out_hbm.at[pl.ds(base, half)], sem
        ).wait()

    return body(lstm, seqlens)


def kernel(lstm, seqlens):
    B, T, D = lstm.shape
    return _last_relevant_sc(lstm, seqlens, B, T, D)


# speedup vs baseline: 1.0471x; 1.0471x over previous
import functools

import jax
import jax.numpy as jnp
from jax.experimental import pallas as pl
from jax.experimental.pallas import tpu as pltpu
from jax.experimental.pallas import tpu_sc as plsc


def _last_relevant_sc(lstm, seqlens, B, T, D):
    mesh = plsc.ScalarSubcoreMesh(axis_name="c", num_cores=1)

    @functools.partial(
        pl.kernel,
        mesh=mesh,
        out_type=jax.ShapeDtypeStruct((B, D), jnp.float32),
        scratch_types=[
            pltpu.SMEM((B,), jnp.int32),
            pltpu.SemaphoreType.DMA,
        ],
    )
    def body(lstm_hbm, seq_hbm, out_hbm, seq_s, sem):
        pltpu.sync_copy(seq_hbm, seq_s)
        for b in range(B):
            t = seq_s[b] - 1
            pltpu.async_copy(lstm_hbm.at[b, t], out_hbm.at[b], sem)
        pltpu.make_async_copy(lstm_hbm.at[0, pl.ds(0, B)], out_hbm, sem).wait()

    return body(lstm, seqlens)


def kernel(lstm, seqlens):
    B, T, D = lstm.shape
    return _last_relevant_sc(lstm, seqlens, B, T, D)


# speedup vs baseline: 1.0750x; 1.0267x over previous
import functools

import jax
import jax.numpy as jnp
from jax.experimental import pallas as pl
from jax.experimental.pallas import tpu as pltpu
from jax.experimental.pallas import tpu_sc as plsc


def _last_relevant_sc(lstm, seqlens, B, T, D):
    mesh = plsc.ScalarSubcoreMesh(axis_name="c", num_cores=1)

    @functools.partial(
        pl.kernel,
        mesh=mesh,
        out_type=jax.ShapeDtypeStruct((B, D), jnp.float32),
        scratch_types=[
            pltpu.SMEM((B,), jnp.int32),
            pltpu.SemaphoreType.DMA,
        ],
    )
    def body(lstm_hbm, seq_hbm, out_hbm, seq_s, sem):
        pltpu.sync_copy(seq_hbm, seq_s)

        def issue(b, carry):
            t = seq_s[b] - 1
            pltpu.async_copy(lstm_hbm.at[b, t], out_hbm.at[b], sem)
            return carry

        jax.lax.fori_loop(0, B, issue, 0)
        pltpu.make_async_copy(lstm_hbm.at[0, pl.ds(0, B)], out_hbm, sem).wait()

    return body(lstm, seqlens)


def kernel(lstm, seqlens):
    B, T, D = lstm.shape
    return _last_relevant_sc(lstm, seqlens, B, T, D)
